# Initial kernel scaffold; baseline (speedup 1.0000x reference)
import jax
import jax.numpy as jnp
from jax.experimental import pallas as pl


def kernel(x, edge_index, batch, W1, as1, ad1, b1, W2, as2, ad2, b2, W3, as3, ad3, b3, g1, be1, g2, be2, g3, be3, Wf, bf, g4, be4, Wl1, bl1, Wl2, bl2):
    raise NotImplementedError("write your pallas kernel here")


# speedup vs baseline: 13.7494x; 13.7494x over previous
import functools
import jax
import jax.numpy as jnp
from jax import lax
from jax.experimental import pallas as pl
from jax.experimental.pallas import tpu as pltpu
from jax.experimental.pallas import tpu_sc as plsc

N = 50000
NPAD = 50176
RPT = NPAD // 16
NH = 25088
NHR = 25600
ZR = 32
G = 16
CH = 128
NTILE = 16

f32 = jnp.float32


def _sc_run(s, n_chunks, srcs, dsts, alS, alD, htabs, utabs, den_hbm,
            alS_v, alD_v, srcb, dstb, dstl, pb, hb, zbuf, sem):
    pltpu.sync_copy(alS, alS_v)
    pltpu.sync_copy(alD, alD_v)


    u0v = (lax.iota(jnp.int32, 16) == 0).astype(f32)

    for f in range(-1, len(htabs)):
        for half in range(2):
            def zb(i, _):
                zbuf[i, pl.ds(0, 16)] = jnp.zeros((16,), f32)
                return 0
            lax.fori_loop(0, ZR, zb, 0)

            def zu(i, _):
                pltpu.sync_copy(
                    zbuf, _sc_run.u_s.at[pl.ds(s * 1600 + i * ZR, ZR)])
                return 0
            lax.fori_loop(0, 1600 // ZR, zu, 0)

            plsc.subcore_barrier()

            def chunk(g0, _):
                base = s * (n_chunks * CH) + g0 * CH
                pltpu.sync_copy(srcs.at[pl.ds(base, CH)], srcb)
                pltpu.sync_copy(dsts.at[pl.ds(base, CH)], dstb)
                for j in range(CH // 16):
                    sv = srcb[pl.ds(j * 16, 16)]
                    dv = dstb[pl.ds(j * 16, 16)]
                    a = plsc.load_gather(alS_v, [sv])
                    b = plsc.load_gather(alD_v, [dv])
                    e = a + b
                    e = jnp.where(e >= 0, e, 0.2 * e)
                    pb[pl.ds(j * 16, 16)] = jnp.exp(e)
                    lv = dv - half * NH
                    m = (lv >= 0) & (lv < NH)
                    dstl[pl.ds(j * 16, 16)] = jnp.where(m, lv, NH)
                if f >= 0:
                    pltpu.async_copy(htabs[f].at[srcb], hb, sem).wait()
                for j in range(CH // 16):
                    pv = pb[pl.ds(j * 16, 16)]
                    for e3 in range(16):
                        r = j * 16 + e3
                        pe = pv[e3]
                        if f >= 0:
                            hb[r, pl.ds(0, 16)] = hb[r, pl.ds(0, 16)] * pe
                        else:
                            hb[r, pl.ds(0, 16)] = u0v * pe
                pltpu.sync_copy(hb, _sc_run.u_s.at[dstl], add=True)
                return 0
            lax.fori_loop(0, n_chunks, chunk, 0)

            plsc.subcore_barrier()

            tgt = den_hbm if f < 0 else utabs[f]

            def du(i, _):
                pltpu.sync_copy(
                    _sc_run.u_s.at[pl.ds(s * 1568 + i * ZR, ZR)], zbuf)
                pltpu.sync_copy(
                    zbuf,
                    tgt.at[pl.ds(half * NH + s * 1568 + i * ZR, ZR)])
                return 0
            lax.fori_loop(0, 1568 // ZR, du, 0)


def make_sc_layer(ncpc, epad):
    n_chunks = epad // (NTILE * CH)
    nt = 2 * ncpc

    def body(*refs):
        (srcs, dsts, alS0, alD0, alS1, alD1) = refs[:6]
        htabs = refs[6:6 + nt]
        utabs = refs[6 + nt:6 + 2 * nt]
        den0, den1 = refs[6 + 2 * nt:8 + 2 * nt]
        (alS_v, alD_v, srcb, dstb, dstl, pb, hb, zbuf, u_s,
         sem) = refs[8 + 2 * nt:]
        _sc_run.u_s = u_s
        c = lax.axis_index("c")
        s = lax.axis_index("s")

        @pl.when(c == 0)
        def _():
            _sc_run(s, n_chunks, srcs, dsts, alS0, alD0, htabs[:ncpc],
                    utabs[:ncpc], den0, alS_v, alD_v, srcb, dstb, dstl, pb,
                    hb, zbuf, sem)

        @pl.when(c == 1)
        def _():
            _sc_run(s, n_chunks, srcs, dsts, alS1, alD1, htabs[ncpc:],
                    utabs[ncpc:], den1, alS_v, alD_v, srcb, dstb, dstl, pb,
                    hb, zbuf, sem)

    out_type = ([jax.ShapeDtypeStruct((NPAD, 16), f32)] * nt
                + [jax.ShapeDtypeStruct((NPAD, 16), f32)] * 2)
    scratch = [
        pltpu.VMEM((NPAD,), f32),
        pltpu.VMEM((NPAD,), f32),
        pltpu.VMEM((CH,), jnp.int32),
        pltpu.VMEM((CH,), jnp.int32),
        pltpu.VMEM((CH,), jnp.int32),
        pltpu.VMEM((CH,), f32),
        pltpu.VMEM((CH, 16), f32),
        pltpu.VMEM((ZR, 16), f32),
        pltpu.VMEM_SHARED((NHR, 16), f32),
        pltpu.SemaphoreType.DMA,
    ]
    mesh = plsc.VectorSubcoreMesh(core_axis_name="c", subcore_axis_name="s",
                                  num_cores=2, num_subcores=NTILE)
    return pl.kernel(body, out_type=out_type, mesh=mesh,
                     scratch_types=scratch,
                     compiler_params=pltpu.CompilerParams(
                         needs_layout_passes=False,
                         use_tc_tiling_on_sc=False))


NB = NPAD // 256


def _rows_mask(i):
    r = lax.broadcasted_iota(jnp.int32, (256, 1), 0) + i * 256
    return r < N


def _mk_o(utab_blocks, d0, d1, b):
    nt = len(utab_blocks)
    ncpc = nt // 2
    parts = []
    for t in range(nt):
        d = d0 if t < ncpc else d1
        parts.append(utab_blocks[t] / d)
    return jnp.concatenate(parts, axis=1) + b


def _den_block(ref):
    return jnp.maximum(ref[...][:, 0:1], 1e-30)


def make_stats(ncpc):
    nt = 2 * ncpc
    F = nt * 16

    def body(*refs):
        utabs = refs[:nt]
        den0, den1, b = refs[nt:nt + 3]
        mu, var = refs[nt + 3:nt + 5]
        accs, accq = refs[nt + 5:]
        i = pl.program_id(0)

        @pl.when(i == 0)
        def _():
            accs[...] = jnp.zeros((1, F), f32)
            accq[...] = jnp.zeros((1, F), f32)

        o = _mk_o([u[...] for u in utabs], _den_block(den0),
                  _den_block(den1), b[...])
        o = jnp.where(_rows_mask(i), o, 0.0)
        accs[...] = accs[...] + o.sum(0)[None, :]
        accq[...] = accq[...] + (o * o).sum(0)[None, :]

        @pl.when(i == NB - 1)
        def _():
            m = accs[...] / N
            mu[...] = m
            var[...] = accq[...] / N - m * m

    ospec = [pl.BlockSpec((256, 16), lambda i: (i, 0))] * nt
    dspec = [pl.BlockSpec((256, 16), lambda i: (i, 0))] * 2
    full = pl.BlockSpec((1, F), lambda i: (0, 0))
    return pl.pallas_call(
        body,
        grid=(NB,),
        in_specs=ospec + dspec + [full],
        out_specs=[full, full],
        out_shape=[jax.ShapeDtypeStruct((1, F), f32)] * 2,
        scratch_shapes=[pltpu.VMEM((1, F), f32)] * 2,
    )


def make_apply(ncpc, ocn):
    nt = 2 * ncpc
    F = nt * 16
    Fn = 2 * ocn
    ntn = Fn // 16

    def body(*refs):
        utabs = refs[:nt]
        den0, den1, b, mu, var, gam, bet, Wn, asn, adn = refs[nt:nt + 10]
        houts = refs[nt + 10:nt + 10 + ntn]
        als0, als1, ald0, ald1 = refs[nt + 10 + ntn:]
        o = _mk_o([u[...] for u in utabs], _den_block(den0),
                  _den_block(den1), b[...])
        xn = (o - mu[...]) * lax.rsqrt(var[...] + 1e-5) * gam[...] + bet[...]
        xn = jnp.maximum(xn, 0.0)
        hn = jnp.dot(xn, Wn[...], preferred_element_type=f32)
        for t in range(ntn):
            houts[t][...] = hn[:, t * 16:(t + 1) * 16]
        for k, (outs, outd) in enumerate(((als0, ald0), (als1, ald1))):
            hk = hn[:, k * ocn:(k + 1) * ocn]
            outs[...] = (hk * asn[k][None, :]).sum(-1).reshape(1, 2, 128)
            outd[...] = (hk * adn[k][None, :]).sum(-1).reshape(1, 2, 128)

    ospec = [pl.BlockSpec((256, 16), lambda i: (i, 0))] * nt
    dspec = [pl.BlockSpec((256, 16), lambda i: (i, 0))] * 2
    fullF = pl.BlockSpec((1, F), lambda i: (0, 0))
    return pl.pallas_call(
        body,
        grid=(NB,),
        in_specs=(ospec + dspec + [fullF] * 5
                  + [pl.BlockSpec((F, Fn), lambda i: (0, 0)),
                     pl.BlockSpec((2, ocn), lambda i: (0, 0)),
                     pl.BlockSpec((2, ocn), lambda i: (0, 0))]),
        out_specs=([pl.BlockSpec((256, 16), lambda i: (i, 0))] * ntn
                   + [pl.BlockSpec((1, 2, 128), lambda i: (i, 0, 0))] * 4),
        out_shape=([jax.ShapeDtypeStruct((NPAD, 16), f32)] * ntn
                   + [jax.ShapeDtypeStruct((NB, 2, 128), f32)] * 4),
    )


def make_final_pool(ncpc):
    nt = 2 * ncpc
    F = nt * 16

    def body(*refs):
        utabs = refs[:nt]
        den0, den1, b, mu, var, gam, bet, batch = refs[nt:nt + 8]
        psum, pcnt = refs[nt + 8:nt + 10]
        accp, accc = refs[nt + 10:]
        i = pl.program_id(0)

        @pl.when(i == 0)
        def _():
            accp[...] = jnp.zeros((G, F), f32)
            accc[...] = jnp.zeros((1, 128), f32)

        o = _mk_o([u[...] for u in utabs], _den_block(den0),
                  _den_block(den1), b[...])
        xn = (o - mu[...]) * lax.rsqrt(var[...] + 1e-5) * gam[...] + bet[...]
        xn = jnp.maximum(xn, 0.0)
        oh = (batch[...] == lax.broadcasted_iota(jnp.int32, (1, G), 1)
              ).astype(f32)
        accp[...] = accp[...] + lax.dot_general(
            oh, xn, (((0,), (0,)), ((), ())), preferred_element_type=f32)
        cnt = oh.sum(0)
        accc[...] = accc[...] + jnp.pad(cnt, (0, 128 - G))[None, :]

        @pl.when(i == NB - 1)
        def _():
            psum[...] = accp[...]
            pcnt[...] = accc[...]

    ospec = [pl.BlockSpec((256, 16), lambda i: (i, 0))] * nt
    dspec = [pl.BlockSpec((256, 16), lambda i: (i, 0))] * 2
    fullF = pl.BlockSpec((1, F), lambda i: (0, 0))
    return pl.pallas_call(
        body,
        grid=(NB,),
        in_specs=(ospec + dspec + [fullF] * 5
                  + [pl.BlockSpec((256, 1), lambda i: (i, 0))]),
        out_specs=[pl.BlockSpec((G, F), lambda i: (0, 0)),
                   pl.BlockSpec((1, 128), lambda i: (0, 0))],
        out_shape=[jax.ShapeDtypeStruct((G, F), f32),
                   jax.ShapeDtypeStruct((1, 128), f32)],
        scratch_shapes=[pltpu.VMEM((G, F), f32), pltpu.VMEM((1, 128), f32)],
    )


def _first_tc(xp, W1p, as1, ad1):
    def body(x_ref, w_ref, as_ref, ad_ref, h00, h01, h10, h11,
             als0, als1, ald0, ald1):
        h = jnp.dot(x_ref[...], w_ref[...], preferred_element_type=f32)
        h00[...] = h[:, 0:16]
        h01[...] = h[:, 16:32]
        h10[...] = h[:, 32:48]
        h11[...] = h[:, 48:64]
        for k, (outs, outd) in enumerate(((als0, ald0), (als1, ald1))):
            hk = h[:, k * 32:(k + 1) * 32]
            outs[...] = (hk * as_ref[k][None, :]).sum(-1).reshape(1, 2, 128)
            outd[...] = (hk * ad_ref[k][None, :]).sum(-1).reshape(1, 2, 128)

    return pl.pallas_call(
        body,
        grid=(NB,),
        in_specs=[pl.BlockSpec((256, 8), lambda i: (i, 0)),
                  pl.BlockSpec((8, 64), lambda i: (0, 0)),
                  pl.BlockSpec((2, 32), lambda i: (0, 0)),
                  pl.BlockSpec((2, 32), lambda i: (0, 0))],
        out_specs=([pl.BlockSpec((256, 16), lambda i: (i, 0))] * 4
                   + [pl.BlockSpec((1, 2, 128), lambda i: (i, 0, 0))] * 4),
        out_shape=([jax.ShapeDtypeStruct((NPAD, 16), f32)] * 4
                   + [jax.ShapeDtypeStruct((NB, 2, 128), f32)] * 4),
    )(xp, W1p, as1, ad1)


def _head_tc(psum, pcnt, Wf, bf, g4, be4, Wl1, bl1, Wl2p, bl2p):
    def body(ps, pc, wf, bf_r, g4_r, be4_r, wl1, bl1_r, wl2, bl2_r,
             pooled_o, logp_o):
        cnt = jnp.maximum(pc[0, :G], 1.0)
        pooled = ps[...] / cnt[:, None]
        pooled_o[...] = pooled
        f0 = jnp.dot(pooled, wf[...], preferred_element_type=f32) + bf_r[...]
        m = f0.mean(0, keepdims=True)
        v = (f0 * f0).mean(0, keepdims=True) - m * m
        f = jnp.maximum((f0 - m) * lax.rsqrt(v + 1e-5) * g4_r[...]
                        + be4_r[...], 0.0)
        f = jnp.maximum(jnp.dot(f, wl1[...], preferred_element_type=f32)
                        + bl1_r[...], 0.0)
        lg = jnp.dot(f, wl2[...], preferred_element_type=f32) + bl2_r[...]
        mx = lg.max(1, keepdims=True)
        lse = jnp.log(jnp.exp(lg - mx).sum(1, keepdims=True)) + mx
        logp_o[...] = lg - lse

    full = lambda s: pl.BlockSpec(s, lambda: tuple(0 for _ in s))
    return pl.pallas_call(
        body,
        in_specs=[full((G, 128)), full((1, 128)), full((128, 32)),
                  full((1, 32)), full((1, 32)), full((1, 32)),
                  full((32, 32)), full((1, 32)), full((32, 128)),
                  full((1, 128))],
        out_specs=[full((G, 128)), full((G, 128))],
        out_shape=[jax.ShapeDtypeStruct((G, 128), f32),
                   jax.ShapeDtypeStruct((G, 128), f32)],
    )(psum, pcnt, Wf, bf, g4, be4, Wl1, bl1, Wl2p, bl2p)


def _row(v, w=None):
    v = v.reshape(1, -1)
    if w is not None and v.shape[1] < w:
        v = jnp.pad(v, ((0, 0), (0, w - v.shape[1])))
    return v


@jax.jit
def kernel(x, edge_index, batch, W1, as1, ad1, b1, W2, as2, ad2, b2,
           W3, as3, ad3, b3, g1, be1, g2, be2, g3, be3, Wf, bf, g4, be4,
           Wl1, bl1, Wl2, bl2):
    n = x.shape[0]
    e = edge_index.shape[1]
    etot = e + n
    epad = ((etot + NTILE * CH - 1) // (NTILE * CH)) * (NTILE * CH)
    loop = jnp.arange(n, dtype=jnp.int32)
    src = jnp.concatenate([edge_index[0], loop,
                           jnp.zeros((epad - etot,), jnp.int32)])
    dst = jnp.concatenate([edge_index[1], loop,
                           jnp.full((epad - etot,), n, jnp.int32)])

    xp = jnp.zeros((NPAD, 8), f32).at[:n, :3].set(x[:, :3])
    W1p = jnp.zeros((8, 64), f32).at[:3].set(W1)

    h1t = _first_tc(xp, W1p, as1, ad1)
    h1 = h1t[:4]
    als0, als1, ald0, ald1 = h1t[4:]

    sc1 = make_sc_layer(2, epad)
    flat = lambda a: a.reshape(-1)
    outs1 = sc1(src, dst, flat(als0), flat(ald0), flat(als1), flat(ald1),
                *h1)
    u1 = outs1[:4]
    d03, d13 = outs1[4], outs1[5]

    mu1, var1 = make_stats(2)(*u1, d03, d13, _row(b1))
    h_tabs2 = make_apply(2, 64)(*u1, d03, d13, _row(b1), mu1, var1,
                                _row(g1), _row(be1), W2, as2, ad2)
    h2 = h_tabs2[:8]
    als0, als1, ald0, ald1 = h_tabs2[8:]

    sc2 = make_sc_layer(4, epad)
    outs2 = sc2(src, dst, flat(als0), flat(ald0), flat(als1), flat(ald1),
                *h2)
    u2 = outs2[:8]
    d03, d13 = outs2[8], outs2[9]

    mu2, var2 = make_stats(4)(*u2, d03, d13, _row(b2))
    h_tabs3 = make_apply(4, 64)(*u2, d03, d13, _row(b2), mu2, var2,
                                _row(g2), _row(be2), W3, as3, ad3)
    h3 = h_tabs3[:8]
    als0, als1, ald0, ald1 = h_tabs3[8:]

    sc3 = sc2
    outs3 = sc3(src, dst, flat(als0), flat(ald0), flat(als1), flat(ald1),
                *h3)
    u3 = outs3[:8]
    d03, d13 = outs3[8], outs3[9]

    mu3, var3 = make_stats(4)(*u3, d03, d13, _row(b3))
    batch2d = jnp.pad(batch.astype(jnp.int32), (0, NPAD - n),
                      constant_values=G).reshape(NPAD, 1)
    psum, pcnt = make_final_pool(4)(*u3, d03, d13, _row(b3), mu3, var3,
                                    _row(g3), _row(be3), batch2d)

    Wl2p = jnp.zeros((32, 128), f32).at[:, :10].set(Wl2)
    bl2p = jnp.full((1, 128), -1e30, f32).at[0, :10].set(bl2)
    pooled, logp = _head_tc(psum, pcnt, Wf, _row(bf), _row(g4), _row(be4),
                            Wl1, _row(bl1), Wl2p, bl2p)
    return pooled, logp[:, :10]


# speedup vs baseline: 14.3878x; 1.0464x over previous
import functools
import jax
import jax.numpy as jnp
from jax import lax
from jax.experimental import pallas as pl
from jax.experimental.pallas import tpu as pltpu
from jax.experimental.pallas import tpu_sc as plsc

N = 50000
NPAD = 50176
RPT = NPAD // 16
NH = 25088
NHR = 25600
ZR = 32
G = 16
CH = 128
NTILE = 16

f32 = jnp.float32


def _sc_run(s, n_chunks, srcs, dsts, alS, alD, htabs, utabs, den_hbm,
            alS_v, alD_v, srcb, dstb, dstl, pb, hb, zbuf, sem):
    pltpu.sync_copy(alS, alS_v)
    pltpu.sync_copy(alD, alD_v)


    u0v = (lax.iota(jnp.int32, 16) == 0).astype(f32)

    for f in range(-1, len(htabs)):
        for half in range(2):
            def zb(i, _):
                zbuf[i, pl.ds(0, 16)] = jnp.zeros((16,), f32)
                return 0
            lax.fori_loop(0, ZR, zb, 0)

            def zu(i, _):
                pltpu.sync_copy(
                    zbuf, _sc_run.u_s.at[pl.ds(s * 1600 + i * ZR, ZR)])
                return 0
            lax.fori_loop(0, 1600 // ZR, zu, 0)

            plsc.subcore_barrier()

            def chunk(g0, _):
                base = s * (n_chunks * CH) + g0 * CH
                pltpu.sync_copy(srcs.at[pl.ds(base, CH)], srcb)
                pltpu.sync_copy(dsts.at[pl.ds(base, CH)], dstb)
                if f >= 0:
                    gat = pltpu.async_copy(htabs[f].at[srcb], hb, sem)
                for j in range(CH // 16):
                    sv = srcb[pl.ds(j * 16, 16)]
                    dv = dstb[pl.ds(j * 16, 16)]
                    a = plsc.load_gather(alS_v, [sv])
                    b = plsc.load_gather(alD_v, [dv])
                    e = a + b
                    e = jnp.where(e >= 0, e, 0.2 * e)
                    pb[pl.ds(j * 16, 16)] = jnp.exp(e)
                    lv = dv - half * NH
                    m = (lv >= 0) & (lv < NH)
                    dstl[pl.ds(j * 16, 16)] = jnp.where(m, lv, NH)
                if f >= 0:
                    gat.wait()
                for j in range(CH // 16):
                    pv = pb[pl.ds(j * 16, 16)]
                    for e3 in range(16):
                        r = j * 16 + e3
                        pe = pv[e3]
                        if f >= 0:
                            hb[r, pl.ds(0, 16)] = hb[r, pl.ds(0, 16)] * pe
                        else:
                            hb[r, pl.ds(0, 16)] = u0v * pe
                pltpu.sync_copy(hb, _sc_run.u_s.at[dstl], add=True)
                return 0
            lax.fori_loop(0, n_chunks, chunk, 0)

            plsc.subcore_barrier()

            tgt = den_hbm if f < 0 else utabs[f]

            def du(i, _):
                pltpu.sync_copy(
                    _sc_run.u_s.at[pl.ds(s * 1568 + i * ZR, ZR)], zbuf)
                pltpu.sync_copy(
                    zbuf,
                    tgt.at[pl.ds(half * NH + s * 1568 + i * ZR, ZR)])
                return 0
            lax.fori_loop(0, 1568 // ZR, du, 0)


def make_sc_layer(ncpc, epad):
    n_chunks = epad // (NTILE * CH)
    nt = 2 * ncpc

    def body(*refs):
        (srcs, dsts, alS0, alD0, alS1, alD1) = refs[:6]
        htabs = refs[6:6 + nt]
        utabs = refs[6 + nt:6 + 2 * nt]
        den0, den1 = refs[6 + 2 * nt:8 + 2 * nt]
        (alS_v, alD_v, srcb, dstb, dstl, pb, hb, zbuf, u_s,
         sem) = refs[8 + 2 * nt:]
        _sc_run.u_s = u_s
        c = lax.axis_index("c")
        s = lax.axis_index("s")

        @pl.when(c == 0)
        def _():
            _sc_run(s, n_chunks, srcs, dsts, alS0, alD0, htabs[:ncpc],
                    utabs[:ncpc], den0, alS_v, alD_v, srcb, dstb, dstl, pb,
                    hb, zbuf, sem)

        @pl.when(c == 1)
        def _():
            _sc_run(s, n_chunks, srcs, dsts, alS1, alD1, htabs[ncpc:],
                    utabs[ncpc:], den1, alS_v, alD_v, srcb, dstb, dstl, pb,
                    hb, zbuf, sem)

    out_type = ([jax.ShapeDtypeStruct((NPAD, 16), f32)] * nt
                + [jax.ShapeDtypeStruct((NPAD, 16), f32)] * 2)
    scratch = [
        pltpu.VMEM((NPAD,), f32),
        pltpu.VMEM((NPAD,), f32),
        pltpu.VMEM((CH,), jnp.int32),
        pltpu.VMEM((CH,), jnp.int32),
        pltpu.VMEM((CH,), jnp.int32),
        pltpu.VMEM((CH,), f32),
        pltpu.VMEM((CH, 16), f32),
        pltpu.VMEM((ZR, 16), f32),
        pltpu.VMEM_SHARED((NHR, 16), f32),
        pltpu.SemaphoreType.DMA,
    ]
    mesh = plsc.VectorSubcoreMesh(core_axis_name="c", subcore_axis_name="s",
                                  num_cores=2, num_subcores=NTILE)
    return pl.kernel(body, out_type=out_type, mesh=mesh,
                     scratch_types=scratch,
                     compiler_params=pltpu.CompilerParams(
                         needs_layout_passes=False,
                         use_tc_tiling_on_sc=False))


NB = NPAD // 256


def _rows_mask(i):
    r = lax.broadcasted_iota(jnp.int32, (256, 1), 0) + i * 256
    return r < N


def _mk_o(utab_blocks, d0, d1, b):
    nt = len(utab_blocks)
    ncpc = nt // 2
    parts = []
    for t in range(nt):
        d = d0 if t < ncpc else d1
        parts.append(utab_blocks[t] / d)
    return jnp.concatenate(parts, axis=1) + b


def _den_block(ref):
    return jnp.maximum(ref[...][:, 0:1], 1e-30)


def make_stats(ncpc):
    nt = 2 * ncpc
    F = nt * 16

    def body(*refs):
        utabs = refs[:nt]
        den0, den1, b = refs[nt:nt + 3]
        mu, var = refs[nt + 3:nt + 5]
        accs, accq = refs[nt + 5:]
        i = pl.program_id(0)

        @pl.when(i == 0)
        def _():
            accs[...] = jnp.zeros((1, F), f32)
            accq[...] = jnp.zeros((1, F), f32)

        o = _mk_o([u[...] for u in utabs], _den_block(den0),
                  _den_block(den1), b[...])
        o = jnp.where(_rows_mask(i), o, 0.0)
        accs[...] = accs[...] + o.sum(0)[None, :]
        accq[...] = accq[...] + (o * o).sum(0)[None, :]

        @pl.when(i == NB - 1)
        def _():
            m = accs[...] / N
            mu[...] = m
            var[...] = accq[...] / N - m * m

    ospec = [pl.BlockSpec((256, 16), lambda i: (i, 0))] * nt
    dspec = [pl.BlockSpec((256, 16), lambda i: (i, 0))] * 2
    full = pl.BlockSpec((1, F), lambda i: (0, 0))
    return pl.pallas_call(
        body,
        grid=(NB,),
        in_specs=ospec + dspec + [full],
        out_specs=[full, full],
        out_shape=[jax.ShapeDtypeStruct((1, F), f32)] * 2,
        scratch_shapes=[pltpu.VMEM((1, F), f32)] * 2,
    )


def make_apply(ncpc, ocn):
    nt = 2 * ncpc
    F = nt * 16
    Fn = 2 * ocn
    ntn = Fn // 16

    def body(*refs):
        utabs = refs[:nt]
        den0, den1, b, mu, var, gam, bet, Wn, asn, adn = refs[nt:nt + 10]
        houts = refs[nt + 10:nt + 10 + ntn]
        als0, als1, ald0, ald1 = refs[nt + 10 + ntn:]
        o = _mk_o([u[...] for u in utabs], _den_block(den0),
                  _den_block(den1), b[...])
        xn = (o - mu[...]) * lax.rsqrt(var[...] + 1e-5) * gam[...] + bet[...]
        xn = jnp.maximum(xn, 0.0)
        hn = jnp.dot(xn, Wn[...], preferred_element_type=f32)
        for t in range(ntn):
            houts[t][...] = hn[:, t * 16:(t + 1) * 16]
        for k, (outs, outd) in enumerate(((als0, ald0), (als1, ald1))):
            hk = hn[:, k * ocn:(k + 1) * ocn]
            outs[...] = (hk * asn[k][None, :]).sum(-1).reshape(1, 2, 128)
            outd[...] = (hk * adn[k][None, :]).sum(-1).reshape(1, 2, 128)

    ospec = [pl.BlockSpec((256, 16), lambda i: (i, 0))] * nt
    dspec = [pl.BlockSpec((256, 16), lambda i: (i, 0))] * 2
    fullF = pl.BlockSpec((1, F), lambda i: (0, 0))
    return pl.pallas_call(
        body,
        grid=(NB,),
        in_specs=(ospec + dspec + [fullF] * 5
                  + [pl.BlockSpec((F, Fn), lambda i: (0, 0)),
                     pl.BlockSpec((2, ocn), lambda i: (0, 0)),
                     pl.BlockSpec((2, ocn), lambda i: (0, 0))]),
        out_specs=([pl.BlockSpec((256, 16), lambda i: (i, 0))] * ntn
                   + [pl.BlockSpec((1, 2, 128), lambda i: (i, 0, 0))] * 4),
        out_shape=([jax.ShapeDtypeStruct((NPAD, 16), f32)] * ntn
                   + [jax.ShapeDtypeStruct((NB, 2, 128), f32)] * 4),
    )


def make_final_pool(ncpc):
    nt = 2 * ncpc
    F = nt * 16

    def body(*refs):
        utabs = refs[:nt]
        den0, den1, b, mu, var, gam, bet, batch = refs[nt:nt + 8]
        psum, pcnt = refs[nt + 8:nt + 10]
        accp, accc = refs[nt + 10:]
        i = pl.program_id(0)

        @pl.when(i == 0)
        def _():
            accp[...] = jnp.zeros((G, F), f32)
            accc[...] = jnp.zeros((1, 128), f32)

        o = _mk_o([u[...] for u in utabs], _den_block(den0),
                  _den_block(den1), b[...])
        xn = (o - mu[...]) * lax.rsqrt(var[...] + 1e-5) * gam[...] + bet[...]
        xn = jnp.maximum(xn, 0.0)
        oh = (batch[...] == lax.broadcasted_iota(jnp.int32, (1, G), 1)
              ).astype(f32)
        accp[...] = accp[...] + lax.dot_general(
            oh, xn, (((0,), (0,)), ((), ())), preferred_element_type=f32)
        cnt = oh.sum(0)
        accc[...] = accc[...] + jnp.pad(cnt, (0, 128 - G))[None, :]

        @pl.when(i == NB - 1)
        def _():
            psum[...] = accp[...]
            pcnt[...] = accc[...]

    ospec = [pl.BlockSpec((256, 16), lambda i: (i, 0))] * nt
    dspec = [pl.BlockSpec((256, 16), lambda i: (i, 0))] * 2
    fullF = pl.BlockSpec((1, F), lambda i: (0, 0))
    return pl.pallas_call(
        body,
        grid=(NB,),
        in_specs=(ospec + dspec + [fullF] * 5
                  + [pl.BlockSpec((256, 1), lambda i: (i, 0))]),
        out_specs=[pl.BlockSpec((G, F), lambda i: (0, 0)),
                   pl.BlockSpec((1, 128), lambda i: (0, 0))],
        out_shape=[jax.ShapeDtypeStruct((G, F), f32),
                   jax.ShapeDtypeStruct((1, 128), f32)],
        scratch_shapes=[pltpu.VMEM((G, F), f32), pltpu.VMEM((1, 128), f32)],
    )


def _first_tc(xp, W1p, as1, ad1):
    def body(x_ref, w_ref, as_ref, ad_ref, h00, h01, h10, h11,
             als0, als1, ald0, ald1):
        h = jnp.dot(x_ref[...], w_ref[...], preferred_element_type=f32)
        h00[...] = h[:, 0:16]
        h01[...] = h[:, 16:32]
        h10[...] = h[:, 32:48]
        h11[...] = h[:, 48:64]
        for k, (outs, outd) in enumerate(((als0, ald0), (als1, ald1))):
            hk = h[:, k * 32:(k + 1) * 32]
            outs[...] = (hk * as_ref[k][None, :]).sum(-1).reshape(1, 2, 128)
            outd[...] = (hk * ad_ref[k][None, :]).sum(-1).reshape(1, 2, 128)

    return pl.pallas_call(
        body,
        grid=(NB,),
        in_specs=[pl.BlockSpec((256, 8), lambda i: (i, 0)),
                  pl.BlockSpec((8, 64), lambda i: (0, 0)),
                  pl.BlockSpec((2, 32), lambda i: (0, 0)),
                  pl.BlockSpec((2, 32), lambda i: (0, 0))],
        out_specs=([pl.BlockSpec((256, 16), lambda i: (i, 0))] * 4
                   + [pl.BlockSpec((1, 2, 128), lambda i: (i, 0, 0))] * 4),
        out_shape=([jax.ShapeDtypeStruct((NPAD, 16), f32)] * 4
                   + [jax.ShapeDtypeStruct((NB, 2, 128), f32)] * 4),
    )(xp, W1p, as1, ad1)


def _head_tc(psum, pcnt, Wf, bf, g4, be4, Wl1, bl1, Wl2p, bl2p):
    def body(ps, pc, wf, bf_r, g4_r, be4_r, wl1, bl1_r, wl2, bl2_r,
             pooled_o, logp_o):
        cnt = jnp.maximum(pc[0, :G], 1.0)
        pooled = ps[...] / cnt[:, None]
        pooled_o[...] = pooled
        f0 = jnp.dot(pooled, wf[...], preferred_element_type=f32) + bf_r[...]
        m = f0.mean(0, keepdims=True)
        v = (f0 * f0).mean(0, keepdims=True) - m * m
        f = jnp.maximum((f0 - m) * lax.rsqrt(v + 1e-5) * g4_r[...]
                        + be4_r[...], 0.0)
        f = jnp.maximum(jnp.dot(f, wl1[...], preferred_element_type=f32)
                        + bl1_r[...], 0.0)
        lg = jnp.dot(f, wl2[...], preferred_element_type=f32) + bl2_r[...]
        mx = lg.max(1, keepdims=True)
        lse = jnp.log(jnp.exp(lg - mx).sum(1, keepdims=True)) + mx
        logp_o[...] = lg - lse

    full = lambda s: pl.BlockSpec(s, lambda: tuple(0 for _ in s))
    return pl.pallas_call(
        body,
        in_specs=[full((G, 128)), full((1, 128)), full((128, 32)),
                  full((1, 32)), full((1, 32)), full((1, 32)),
                  full((32, 32)), full((1, 32)), full((32, 128)),
                  full((1, 128))],
        out_specs=[full((G, 128)), full((G, 128))],
        out_shape=[jax.ShapeDtypeStruct((G, 128), f32),
                   jax.ShapeDtypeStruct((G, 128), f32)],
    )(psum, pcnt, Wf, bf, g4, be4, Wl1, bl1, Wl2p, bl2p)


def _row(v, w=None):
    v = v.reshape(1, -1)
    if w is not None and v.shape[1] < w:
        v = jnp.pad(v, ((0, 0), (0, w - v.shape[1])))
    return v


@jax.jit
def kernel(x, edge_index, batch, W1, as1, ad1, b1, W2, as2, ad2, b2,
           W3, as3, ad3, b3, g1, be1, g2, be2, g3, be3, Wf, bf, g4, be4,
           Wl1, bl1, Wl2, bl2):
    n = x.shape[0]
    e = edge_index.shape[1]
    etot = e + n
    epad = ((etot + NTILE * CH - 1) // (NTILE * CH)) * (NTILE * CH)
    loop = jnp.arange(n, dtype=jnp.int32)
    src = jnp.concatenate([edge_index[0], loop,
                           jnp.zeros((epad - etot,), jnp.int32)])
    dst = jnp.concatenate([edge_index[1], loop,
                           jnp.full((epad - etot,), n, jnp.int32)])

    xp = jnp.zeros((NPAD, 8), f32).at[:n, :3].set(x[:, :3])
    W1p = jnp.zeros((8, 64), f32).at[:3].set(W1)

    h1t = _first_tc(xp, W1p, as1, ad1)
    h1 = h1t[:4]
    als0, als1, ald0, ald1 = h1t[4:]

    sc1 = make_sc_layer(2, epad)
    flat = lambda a: a.reshape(-1)
    outs1 = sc1(src, dst, flat(als0), flat(ald0), flat(als1), flat(ald1),
                *h1)
    u1 = outs1[:4]
    d03, d13 = outs1[4], outs1[5]

    mu1, var1 = make_stats(2)(*u1, d03, d13, _row(b1))
    h_tabs2 = make_apply(2, 64)(*u1, d03, d13, _row(b1), mu1, var1,
                                _row(g1), _row(be1), W2, as2, ad2)
    h2 = h_tabs2[:8]
    als0, als1, ald0, ald1 = h_tabs2[8:]

    sc2 = make_sc_layer(4, epad)
    outs2 = sc2(src, dst, flat(als0), flat(ald0), flat(als1), flat(ald1),
                *h2)
    u2 = outs2[:8]
    d03, d13 = outs2[8], outs2[9]

    mu2, var2 = make_stats(4)(*u2, d03, d13, _row(b2))
    h_tabs3 = make_apply(4, 64)(*u2, d03, d13, _row(b2), mu2, var2,
                                _row(g2), _row(be2), W3, as3, ad3)
    h3 = h_tabs3[:8]
    als0, als1, ald0, ald1 = h_tabs3[8:]

    sc3 = sc2
    outs3 = sc3(src, dst, flat(als0), flat(ald0), flat(als1), flat(ald1),
                *h3)
    u3 = outs3[:8]
    d03, d13 = outs3[8], outs3[9]

    mu3, var3 = make_stats(4)(*u3, d03, d13, _row(b3))
    batch2d = jnp.pad(batch.astype(jnp.int32), (0, NPAD - n),
                      constant_values=G).reshape(NPAD, 1)
    psum, pcnt = make_final_pool(4)(*u3, d03, d13, _row(b3), mu3, var3,
                                    _row(g3), _row(be3), batch2d)

    Wl2p = jnp.zeros((32, 128), f32).at[:, :10].set(Wl2)
    bl2p = jnp.full((1, 128), -1e30, f32).at[0, :10].set(bl2)
    pooled, logp = _head_tc(psum, pcnt, Wf, _row(bf), _row(g4), _row(be4),
                            Wl1, _row(bl1), Wl2p, bl2p)
    return pooled, logp[:, :10]
